# baseline (speedup 1.0000x reference)
import jax
import jax.numpy as jnp
from jax.experimental import pallas as pl


_PREC = jax.lax.Precision.HIGH


def _conv2d(x, w, b, stride, padding):
    out = jax.lax.conv_general_dilated(
        x, w, (stride, stride), [(padding, padding), (padding, padding)],
        dimension_numbers=('NCHW', 'OIHW', 'NCHW'), precision=_PREC)
    if b is not None:
        out = out + b[None, :, None, None]
    return out


def _conv_transpose2d(x, w, b, stride, padding):
    k = w.shape[2]
    w_c = jnp.transpose(jnp.flip(w, axis=(2, 3)), (1, 0, 2, 3))
    pad = k - 1 - padding
    out = jax.lax.conv_general_dilated(
        x, w_c, (1, 1), [(pad, pad), (pad, pad)], lhs_dilation=(stride, stride),
        dimension_numbers=('NCHW', 'OIHW', 'NCHW'), precision=_PREC)
    return out + b[None, :, None, None]


def _residual_stack(x, blocks):
    for (w1, w2) in blocks:
        h = jax.nn.relu(x)
        h = _conv2d(h, w1, None, 1, 1)
        h = jax.nn.relu(h)
        h = _conv2d(h, w2, None, 1, 0)
        x = x + h
    return jax.nn.relu(x)


_BLK = 512


def _vq_block(z_ref, w_ref, b_ref, emb_ref, q_ref, loss_ref, counts_ref):
    i = pl.program_id(0)
    zb = z_ref[...]
    f = jax.lax.dot_general(
        zb, w_ref[...], (((1,), (0,)), ((), ())),
        preferred_element_type=jnp.float32) + b_ref[...]
    e = emb_ref[...]
    emb_sq = jnp.sum(e * e, axis=1)
    row_sq = jnp.sum(f * f, axis=1, keepdims=True)
    dot = jax.lax.dot_general(
        f, e, (((1,), (1,)), ((), ())),
        preferred_element_type=jnp.float32)
    d = (row_sq + emb_sq[None, :]) - 2.0 * dot
    minv = jnp.min(d, axis=1, keepdims=True)
    ids = jax.lax.broadcasted_iota(jnp.int32, d.shape, 1)
    idx = jnp.min(jnp.where(d == minv, ids, jnp.int32(1 << 30)), axis=1)
    oh = (ids == idx[:, None]).astype(jnp.float32)
    q = jax.lax.dot_general(
        oh, e, (((1,), (0,)), ((), ())),
        preferred_element_type=jnp.float32)
    q_ref[...] = q
    diff = q - f

    @pl.when(i == 0)
    def _init():
        loss_ref[...] = jnp.zeros_like(loss_ref)
        counts_ref[...] = jnp.zeros_like(counts_ref)

    loss_ref[...] += jnp.sum(diff * diff)[None, None]
    counts_ref[...] += jnp.sum(oh, axis=0)[None, :]


def _vq_quantize(z2d, wmat, bias, emb):
    n, c = z2d.shape
    k, d_dim = emb.shape
    grid = n // _BLK
    q, loss_sum, counts = pl.pallas_call(
        _vq_block,
        grid=(grid,),
        in_specs=[
            pl.BlockSpec((_BLK, c), lambda i: (i, 0)),
            pl.BlockSpec((c, d_dim), lambda i: (0, 0)),
            pl.BlockSpec((1, d_dim), lambda i: (0, 0)),
            pl.BlockSpec((k, d_dim), lambda i: (0, 0)),
        ],
        out_specs=[
            pl.BlockSpec((_BLK, d_dim), lambda i: (i, 0)),
            pl.BlockSpec((1, 1), lambda i: (0, 0)),
            pl.BlockSpec((1, k), lambda i: (0, 0)),
        ],
        out_shape=[
            jax.ShapeDtypeStruct((n, d_dim), jnp.float32),
            jax.ShapeDtypeStruct((1, 1), jnp.float32),
            jax.ShapeDtypeStruct((1, k), jnp.float32),
        ],
    )(z2d, wmat, bias, emb)
    return q, loss_sum[0, 0], counts[0]


def kernel(x, enc_w1, enc_b1, enc_w2, enc_b2, enc_w3, enc_b3, enc_r1_w1,
           enc_r1_w2, enc_r2_w1, enc_r2_w2, prevq_w, prevq_b, emb, dec_w1,
           dec_b1, dec_r1_w1, dec_r1_w2, dec_r2_w1, dec_r2_w2, dec_t1_w,
           dec_t1_b, dec_t2_w, dec_t2_b):
    z = jax.nn.relu(_conv2d(x, enc_w1, enc_b1, 2, 1))
    z = jax.nn.relu(_conv2d(z, enc_w2, enc_b2, 2, 1))
    z = _conv2d(z, enc_w3, enc_b3, 1, 1)
    z = _residual_stack(z, [(enc_r1_w1, enc_r1_w2), (enc_r2_w1, enc_r2_w2)])
    zp = jnp.transpose(z, (0, 2, 3, 1))
    z2d = zp.reshape(-1, zp.shape[-1])
    wmat = prevq_w[:, :, 0, 0].T

    q_flat, loss_sum, counts = _vq_quantize(z2d, wmat, prevq_b[None, :], emb)

    n_rows = z2d.shape[0]
    mse = loss_sum / (n_rows * emb.shape[1])
    loss = mse + 0.25 * mse
    avg_probs = counts / n_rows
    perplexity = jnp.exp(-jnp.sum(avg_probs * jnp.log(avg_probs + 1e-10)))

    quantized = q_flat.reshape(zp.shape[:3] + (emb.shape[1],))
    q = jnp.transpose(quantized, (0, 3, 1, 2))
    h = _conv2d(q, dec_w1, dec_b1, 1, 1)
    h = _residual_stack(h, [(dec_r1_w1, dec_r1_w2), (dec_r2_w1, dec_r2_w2)])
    h = jax.nn.relu(_conv_transpose2d(h, dec_t1_w, dec_t1_b, 2, 1))
    x_recon = _conv_transpose2d(h, dec_t2_w, dec_t2_b, 2, 1)
    return (loss, x_recon, perplexity)


# speedup vs baseline: 2.6300x; 2.6300x over previous
import jax
import jax.numpy as jnp
from jax.experimental import pallas as pl
from jax.experimental.pallas import tpu as pltpu


def _conv2d(x, w, b, stride, padding):
    out = jax.lax.conv_general_dilated(
        x, w, (stride, stride), [(padding, padding), (padding, padding)],
        dimension_numbers=('NCHW', 'OIHW', 'NCHW'))
    if b is not None:
        out = out + b[None, :, None, None]
    return out


def _conv_transpose2d(x, w, b, stride, padding):
    k = w.shape[2]
    w_c = jnp.transpose(jnp.flip(w, axis=(2, 3)), (1, 0, 2, 3))
    pad = k - 1 - padding
    out = jax.lax.conv_general_dilated(
        x, w_c, (1, 1), [(pad, pad), (pad, pad)], lhs_dilation=(stride, stride),
        dimension_numbers=('NCHW', 'OIHW', 'NCHW'))
    return out + b[None, :, None, None]


def _residual_stack(x, blocks):
    for (w1, w2) in blocks:
        h = jax.nn.relu(x)
        h = _conv2d(h, w1, None, 1, 1)
        h = jax.nn.relu(h)
        h = _conv2d(h, w2, None, 1, 0)
        x = x + h
    return jax.nn.relu(x)


_BLK = 1568


def _vq_block(z_ref, w_ref, b_ref, emb_ref, q_ref, loss_ref, counts_ref,
              esq_ref):
    i = pl.program_id(0)
    e = emb_ref[...]

    @pl.when(i == 0)
    def _init():
        loss_ref[...] = jnp.zeros_like(loss_ref)
        counts_ref[...] = jnp.zeros_like(counts_ref)
        esq_ref[...] = jnp.sum(e * e, axis=1)[None, :]

    zb = z_ref[...]
    f = jax.lax.dot_general(
        zb, w_ref[...], (((1,), (0,)), ((), ())),
        preferred_element_type=jnp.float32) + b_ref[...]
    row_sq = jnp.sum(f * f, axis=1, keepdims=True)
    dot2 = jax.lax.dot_general(
        f * (-2.0), e, (((1,), (1,)), ((), ())),
        preferred_element_type=jnp.float32)
    d = (row_sq + esq_ref[...]) + dot2
    minv = jnp.min(d, axis=1, keepdims=True)
    ids = jax.lax.broadcasted_iota(jnp.int32, d.shape, 1)
    idx = jnp.min(jnp.where(d == minv, ids, jnp.int32(1 << 30)), axis=1)
    oh = jnp.where(ids == idx[:, None], 1.0, 0.0)
    q = jax.lax.dot_general(
        oh, e, (((1,), (0,)), ((), ())),
        preferred_element_type=jnp.float32)
    q_ref[...] = q
    diff = q - f

    loss_ref[...] += jnp.sum(diff * diff)[None, None]
    counts_ref[...] += jax.lax.dot_general(
        jnp.ones((1, oh.shape[0]), jnp.float32), oh,
        (((1,), (0,)), ((), ())), preferred_element_type=jnp.float32)


def _vq_quantize(z2d, wmat, bias, emb):
    n, c = z2d.shape
    k, d_dim = emb.shape
    grid = n // _BLK
    q, loss_sum, counts = pl.pallas_call(
        _vq_block,
        grid=(grid,),
        in_specs=[
            pl.BlockSpec((_BLK, c), lambda i: (i, 0)),
            pl.BlockSpec((c, d_dim), lambda i: (0, 0)),
            pl.BlockSpec((1, d_dim), lambda i: (0, 0)),
            pl.BlockSpec((k, d_dim), lambda i: (0, 0)),
        ],
        out_specs=[
            pl.BlockSpec((_BLK, d_dim), lambda i: (i, 0)),
            pl.BlockSpec((1, 1), lambda i: (0, 0)),
            pl.BlockSpec((1, k), lambda i: (0, 0)),
        ],
        out_shape=[
            jax.ShapeDtypeStruct((n, d_dim), jnp.float32),
            jax.ShapeDtypeStruct((1, 1), jnp.float32),
            jax.ShapeDtypeStruct((1, k), jnp.float32),
        ],
        scratch_shapes=[pltpu.VMEM((1, k), jnp.float32)],
    )(z2d, wmat, bias, emb)
    return q, loss_sum[0, 0], counts[0]


def kernel(x, enc_w1, enc_b1, enc_w2, enc_b2, enc_w3, enc_b3, enc_r1_w1,
           enc_r1_w2, enc_r2_w1, enc_r2_w2, prevq_w, prevq_b, emb, dec_w1,
           dec_b1, dec_r1_w1, dec_r1_w2, dec_r2_w1, dec_r2_w2, dec_t1_w,
           dec_t1_b, dec_t2_w, dec_t2_b):
    z = jax.nn.relu(_conv2d(x, enc_w1, enc_b1, 2, 1))
    z = jax.nn.relu(_conv2d(z, enc_w2, enc_b2, 2, 1))
    z = _conv2d(z, enc_w3, enc_b3, 1, 1)
    z = _residual_stack(z, [(enc_r1_w1, enc_r1_w2), (enc_r2_w1, enc_r2_w2)])
    zp = jnp.transpose(z, (0, 2, 3, 1))
    z2d = zp.reshape(-1, zp.shape[-1])
    wmat = prevq_w[:, :, 0, 0].T

    q_flat, loss_sum, counts = _vq_quantize(z2d, wmat, prevq_b[None, :], emb)

    n_rows = z2d.shape[0]
    mse = loss_sum / (n_rows * emb.shape[1])
    loss = mse + 0.25 * mse
    avg_probs = counts / n_rows
    perplexity = jnp.exp(-jnp.sum(avg_probs * jnp.log(avg_probs + 1e-10)))

    quantized = q_flat.reshape(zp.shape[:3] + (emb.shape[1],))
    q = jnp.transpose(quantized, (0, 3, 1, 2))
    h = _conv2d(q, dec_w1, dec_b1, 1, 1)
    h = _residual_stack(h, [(dec_r1_w1, dec_r1_w2), (dec_r2_w1, dec_r2_w2)])
    h = jax.nn.relu(_conv_transpose2d(h, dec_t1_w, dec_t1_b, 2, 1))
    x_recon = _conv_transpose2d(h, dec_t2_w, dec_t2_b, 2, 1)
    return (loss, x_recon, perplexity)
